# baseline (speedup 1.0000x reference)
import functools

import jax
import jax.numpy as jnp
from jax import lax
from jax.experimental import pallas as pl
from jax.experimental.pallas import tpu as pltpu
from jax.experimental.pallas import tpu_sc as plsc

N = 5000
C = 80
THR = 0.2
IOU_THR = 0.2
K = 100

L = 16
NS = 16
ROWS = 320
NV = ROWS // L
NP = NS * ROWS
NEG = -1e30
BIG = 1 << 30
BO_PAD = 448
SC_PAD = 112


def _nms_kernel(sc_hbm, bx_hbm,
                bo_hbm, so_hbm, lo_hbm,
                scv, bxv, x1v, y1v, x2v, y2v, arv, candv, labv,
                pubv, rbv, obv, osv, olv, sem, shared):
    sid = lax.axis_index("s")
    rb = jnp.minimum(sid * ROWS, N - ROWS)
    base = sid * ROWS
    lanes = lax.iota(jnp.int32, L)
    zf = jnp.zeros((L,), jnp.float32)
    zi = jnp.zeros((L,), jnp.int32)
    negv = jnp.full((L,), NEG, jnp.float32)
    bigv = jnp.full((L,), BIG, jnp.int32)

    d1 = pltpu.async_copy(sc_hbm.at[pl.ds(rb * C, ROWS * C)], scv, sem)
    d2 = pltpu.async_copy(bx_hbm.at[pl.ds(rb * 4, ROWS * 4)], bxv, sem)

    def zero_b(j, _):
        obv[pl.ds(j * L, L)] = zf
        return 0
    lax.fori_loop(0, BO_PAD // L, zero_b, 0)

    def zero_s(j, _):
        osv[pl.ds(j * L, L)] = zf
        olv[pl.ds(j * L, L)] = zi
        return 0
    lax.fori_loop(0, SC_PAD // L, zero_s, 0)

    d1.wait()
    d2.wait()

    def top2_update(st, v, idx):
        bS1, bI1, bS2, bI2 = st
        take1 = v > bS1
        take2 = (v > bS2) & (~take1)
        nS2 = jnp.where(take1, bS1, jnp.where(take2, v, bS2))
        nI2 = jnp.where(take1, bI1, jnp.where(take2, idx, bI2))
        nS1 = jnp.where(take1, v, bS1)
        nI1 = jnp.where(take1, idx, bI1)
        return (nS1, nI1, nS2, nI2)

    def grp(g, _):
        row_off = g * (L * C) + lanes * C

        def cls(c8, MA):
            M, A = MA
            for dc in range(8):
                c = c8 * 8 + dc
                v = plsc.load_gather(scv, [row_off + c])
                take = v > M
                M = jnp.where(take, v, M)
                A = jnp.where(take, jnp.full((L,), c, jnp.int32), A)
            return (M, A)

        M, A = lax.fori_loop(0, C // 8, cls,
                             (jnp.full((L,), NEG, jnp.float32), zi))
        sl = pl.ds(g * L, L)
        bo4 = (g * L + lanes) * 4
        x1 = plsc.load_gather(bxv, [bo4])
        y1 = plsc.load_gather(bxv, [bo4 + 1])
        x2 = plsc.load_gather(bxv, [bo4 + 2])
        y2 = plsc.load_gather(bxv, [bo4 + 3])
        x1v[sl] = x1
        y1v[sl] = y1
        x2v[sl] = x2
        y2v[sl] = y2
        arv[sl] = (x2 - x1) * (y2 - y1)
        validrow = (rb + g * L + lanes) >= base
        candv[sl] = jnp.where(validrow & (M > THR), M, negv)
        labv[sl] = A
        return 0
    lax.fori_loop(0, NV, grp, 0)

    def cond(st):
        return st[1]

    def body(st):
        k0, _ = st

        def scan(j2, nst):
            for dj in range(2):
                j = j2 * 2 + dj
                nst = top2_update(nst, candv[pl.ds(j * L, L)],
                                  jnp.full((L,), j * L, jnp.int32) + lanes)
            return nst

        bS1, bI1, bS2, bI2 = lax.fori_loop(
            0, NV // 2, scan,
            (jnp.full((L,), NEG, jnp.float32), bigv,
             jnp.full((L,), NEG, jnp.float32), bigv))

        M1 = jnp.max(bS1)
        li1 = jnp.min(jnp.where(bS1 == M1, bI1, bigv))
        match = (bS1 == M1) & (bI1 == li1)
        bS1m = jnp.where(match, bS2, bS1)
        bI1m = jnp.where(match, bI2, bI1)
        M2 = jnp.max(bS1m)
        li2 = jnp.min(jnp.where(bS1m == M2, bI1m, bigv))
        li1c = jnp.clip(li1, 0, ROWS - 1)
        li2c = jnp.clip(li2, 0, ROWS - 1)

        l1v = jnp.full((L,), li1c, jnp.int32)
        l2v = jnp.full((L,), li2c, jnp.int32)
        a_x1 = plsc.load_gather(x1v, [l1v])
        a_y1 = plsc.load_gather(y1v, [l1v])
        a_x2 = plsc.load_gather(x2v, [l1v])
        a_y2 = plsc.load_gather(y2v, [l1v])
        a_ar = plsc.load_gather(arv, [l1v])
        a_lb = plsc.load_gather(labv, [l1v])
        b_x1 = plsc.load_gather(x1v, [l2v])
        b_y1 = plsc.load_gather(y1v, [l2v])
        b_x2 = plsc.load_gather(x2v, [l2v])
        b_y2 = plsc.load_gather(y2v, [l2v])
        b_ar = plsc.load_gather(arv, [l2v])
        b_lb = plsc.load_gather(labv, [l2v])

        pub = jnp.full((L,), M1, jnp.float32)
        pk1 = jnp.full((L,), sid * 65536 + li1c, jnp.int32)
        pk2 = jnp.full((L,), sid * 65536 + li2c, jnp.int32)
        pub = jnp.where(lanes == 1, plsc.bitcast(pk1, jnp.float32), pub)
        pub = jnp.where(lanes == 2, a_x1, pub)
        pub = jnp.where(lanes == 3, a_y1, pub)
        pub = jnp.where(lanes == 4, a_x2, pub)
        pub = jnp.where(lanes == 5, a_y2, pub)
        pub = jnp.where(lanes == 6, plsc.bitcast(a_lb, jnp.float32), pub)
        pub = jnp.where(lanes == 7, a_ar, pub)
        pub = jnp.where(lanes == 8, jnp.full((L,), M2, jnp.float32), pub)
        pub = jnp.where(lanes == 9, plsc.bitcast(pk2, jnp.float32), pub)
        pub = jnp.where(lanes == 10, b_x1, pub)
        pub = jnp.where(lanes == 11, b_y1, pub)
        pub = jnp.where(lanes == 12, b_x2, pub)
        pub = jnp.where(lanes == 13, b_y2, pub)
        pub = jnp.where(lanes == 14, plsc.bitcast(b_lb, jnp.float32), pub)
        pub = jnp.where(lanes == 15, b_ar, pub)
        pubv[...] = pub

        par = lax.rem(k0, 2)
        pltpu.sync_copy(pubv, shared.at[pl.ds(par * (NS * L) + sid * L, L)])
        plsc.subcore_barrier()
        pltpu.sync_copy(shared.at[pl.ds(par * (NS * L), NS * L)], rbv)

        def col(c):
            return plsc.load_gather(rbv, [lanes * L + c])

        s_a = col(0)
        p_a = plsc.bitcast(col(1), jnp.int32)
        ax1 = col(2)
        ay1 = col(3)
        ax2 = col(4)
        ay2 = col(5)
        aar = col(7)
        s_b = col(8)
        p_b = plsc.bitcast(col(9), jnp.int32)
        bx1 = col(10)
        by1 = col(11)
        bx2 = col(12)
        by2 = col(13)
        bar = col(15)

        dead_a0 = ~(s_a > -1e29)
        dead_b0 = ~(s_b > -1e29)

        def ex_cond(est):
            return est[0]

        def ex_body(est):
            _, k, dead_a, dead_b = est

            ma = jnp.where(dead_a, negv, s_a)
            mb = jnp.where(dead_b, negv, s_b)
            M = jnp.max(jnp.maximum(ma, mb))
            Mv = jnp.full((L,), M, jnp.float32)
            pa = jnp.where((~dead_a) & (s_a == Mv), p_a, bigv)
            pb = jnp.where((~dead_b) & (s_b == Mv), p_b, bigv)
            pw = jnp.min(jnp.minimum(pa, pb))
            pwv = jnp.full((L,), pw, jnp.int32)
            got = M > -1e29
            bothdead = dead_a & dead_b
            outrank = (s_b > Mv) | ((s_b == Mv) & (p_b < pwv))
            viol = jnp.max((bothdead & outrank).astype(jnp.int32)) > 0
            accept = got & (~viol) & (k < K)

            is_w_a = (~dead_a) & (s_a == Mv) & (p_a == pwv)
            is_w_b = (~dead_b) & (s_b == Mv) & (p_b == pwv)
            from_b = jnp.max(is_w_b.astype(jnp.int32)) > 0

            wsid = jnp.clip(jnp.right_shift(pw, 16), 0, NS - 1)
            wrb = jnp.minimum(wsid * ROWS, N - ROWS)
            srow = wsid * L + jnp.where(from_b, 8, 0)
            sr = jnp.full((L,), srow, jnp.int32)
            X1v = plsc.load_gather(rbv, [sr + 2])
            Y1v = plsc.load_gather(rbv, [sr + 3])
            X2v = plsc.load_gather(rbv, [sr + 4])
            Y2v = plsc.load_gather(rbv, [sr + 5])
            LBv = plsc.bitcast(plsc.load_gather(rbv, [sr + 6]), jnp.int32)
            WAv = plsc.load_gather(rbv, [sr + 7])
            iwv = jnp.full((L,), (pw & 0xFFFF) + wrb, jnp.int32)
            acc_v = jnp.full((L,), accept, jnp.bool_)

            def board_iou(ex1, ey1, ex2, ey2, ear):
                xx1 = jnp.maximum(ex1, X1v)
                yy1 = jnp.maximum(ey1, Y1v)
                xx2 = jnp.minimum(ex2, X2v)
                yy2 = jnp.minimum(ey2, Y2v)
                inter = (jnp.maximum(xx2 - xx1, 0.0) *
                         jnp.maximum(yy2 - yy1, 0.0))
                union = WAv + ear - inter
                return inter / (union + 1e-8)

            kill_a = acc_v & (is_w_a | (board_iou(ax1, ay1, ax2, ay2, aar)
                                        >= IOU_THR))
            kill_b = acc_v & (is_w_b | (board_iou(bx1, by1, bx2, by2, bar)
                                        >= IOU_THR))
            dead_a = dead_a | kill_a
            dead_b = dead_b | kill_b

            @pl.when(accept)
            def _():
                def sweep(j4, _):
                    for dj in range(4):
                        j = j4 * 4 + dj
                        sl = pl.ds(j * L, L)
                        xx1 = jnp.maximum(x1v[sl], X1v)
                        yy1 = jnp.maximum(y1v[sl], Y1v)
                        xx2 = jnp.minimum(x2v[sl], X2v)
                        yy2 = jnp.minimum(y2v[sl], Y2v)
                        inter = (jnp.maximum(xx2 - xx1, 0.0) *
                                 jnp.maximum(yy2 - yy1, 0.0))
                        union = WAv + arv[sl] - inter
                        iou = inter / (union + 1e-8)
                        gi = rb + j * L + lanes
                        kill = (iou >= IOU_THR) | (gi == iwv)
                        candv[sl] = jnp.where(kill, negv, candv[sl])
                    return 0
                lax.fori_loop(0, NV // 4, sweep, 0)

            @pl.when(accept & (sid == 0))
            def _():
                bvals = X1v
                bvals = jnp.where(lanes == 1, Y1v, bvals)
                bvals = jnp.where(lanes == 2, X2v, bvals)
                bvals = jnp.where(lanes == 3, Y2v, bvals)
                plsc.store_scatter(obv, [4 * k + lanes], bvals,
                                   mask=lanes < 4)
                kv = jnp.full((L,), k, jnp.int32)
                plsc.store_scatter(osv, [kv], jnp.full((L,), M, jnp.float32),
                                   mask=lanes == 0)
                plsc.store_scatter(olv, [kv], LBv, mask=lanes == 0)

            k = k + accept.astype(jnp.int32)
            return (accept, k, dead_a, dead_b)

        est = lax.while_loop(ex_cond, ex_body,
                             (jnp.bool_(True), k0, dead_a0, dead_b0))
        k1 = est[1]
        cont = (k1 > k0) & (k1 < K)
        return (k1, cont)

    lax.while_loop(cond, body, (jnp.int32(0), jnp.bool_(True)))

    @pl.when(sid == 0)
    def _():
        pltpu.sync_copy(obv, bo_hbm)
        pltpu.sync_copy(osv, so_hbm)
        pltpu.sync_copy(olv, lo_hbm)


@functools.partial(
    pl.kernel,
    out_type=(
        jax.ShapeDtypeStruct((BO_PAD,), jnp.float32),
        jax.ShapeDtypeStruct((SC_PAD,), jnp.float32),
        jax.ShapeDtypeStruct((SC_PAD,), jnp.int32),
    ),
    mesh=plsc.VectorSubcoreMesh(
        core_axis_name="c", subcore_axis_name="s",
        num_cores=1, num_subcores=NS),
    compiler_params=pltpu.CompilerParams(needs_layout_passes=False),
    scratch_types=[
        pltpu.VMEM((ROWS * C,), jnp.float32),
        pltpu.VMEM((ROWS * 4,), jnp.float32),
        pltpu.VMEM((ROWS,), jnp.float32),
        pltpu.VMEM((ROWS,), jnp.float32),
        pltpu.VMEM((ROWS,), jnp.float32),
        pltpu.VMEM((ROWS,), jnp.float32),
        pltpu.VMEM((ROWS,), jnp.float32),
        pltpu.VMEM((ROWS,), jnp.float32),
        pltpu.VMEM((ROWS,), jnp.int32),
        pltpu.VMEM((L,), jnp.float32),
        pltpu.VMEM((NS * L,), jnp.float32),
        pltpu.VMEM((BO_PAD,), jnp.float32),
        pltpu.VMEM((SC_PAD,), jnp.float32),
        pltpu.VMEM((SC_PAD,), jnp.int32),
        pltpu.SemaphoreType.DMA,
        pltpu.VMEM_SHARED((2 * NS * L,), jnp.float32),
    ],
)
def _nms_call(sc_hbm, bx_hbm, bo_hbm, so_hbm, lo_hbm, *scratch):
    _nms_kernel(sc_hbm, bx_hbm, bo_hbm, so_hbm, lo_hbm, *scratch)


@jax.jit
def kernel(boxes, scores):
    bo, so, lo = _nms_call(scores.reshape(-1), boxes.reshape(-1))
    return (bo[:4 * K].reshape(1, K, 4), so[:K][None], lo[:K][None])


# speedup vs baseline: 1.0072x; 1.0072x over previous
import functools

import jax
import jax.numpy as jnp
from jax import lax
from jax.experimental import pallas as pl
from jax.experimental.pallas import tpu as pltpu
from jax.experimental.pallas import tpu_sc as plsc

N = 5000
C = 80
THR = 0.2
IOU_THR = 0.2
K = 100

L = 16
NS = 16
ROWS = 320
NV = ROWS // L
NP = NS * ROWS
NEG = -1e30
BIG = 1 << 30
BO_PAD = 448
SC_PAD = 112


def _nms_kernel(sc_hbm, bx_hbm,
                bo_hbm, so_hbm, lo_hbm,
                scv, bxv, x1v, y1v, x2v, y2v, arv, candv, labv,
                pubv, rbv, obv, osv, olv, sem, shared):
    sid = lax.axis_index("s")
    rb = jnp.minimum(sid * ROWS, N - ROWS)
    base = sid * ROWS
    lanes = lax.iota(jnp.int32, L)
    zf = jnp.zeros((L,), jnp.float32)
    zi = jnp.zeros((L,), jnp.int32)
    negv = jnp.full((L,), NEG, jnp.float32)
    bigv = jnp.full((L,), BIG, jnp.int32)

    d1 = pltpu.async_copy(sc_hbm.at[pl.ds(rb * C, ROWS * C)], scv, sem)
    d2 = pltpu.async_copy(bx_hbm.at[pl.ds(rb * 4, ROWS * 4)], bxv, sem)

    def zero_b(j, _):
        obv[pl.ds(j * L, L)] = zf
        return 0
    lax.fori_loop(0, BO_PAD // L, zero_b, 0)

    def zero_s(j, _):
        osv[pl.ds(j * L, L)] = zf
        olv[pl.ds(j * L, L)] = zi
        return 0
    lax.fori_loop(0, SC_PAD // L, zero_s, 0)

    d1.wait()
    d2.wait()

    def top4_update(st, v, idx):
        S1, I1, S2, I2, S3, I3, S4, I4 = st
        c1 = v > S1
        c2 = (~c1) & (v > S2)
        c3 = (~c1) & (~c2) & (v > S3)
        c4 = (~c1) & (~c2) & (~c3) & (v > S4)
        c12 = c1 | c2
        c123 = c12 | c3
        nS1 = jnp.where(c1, v, S1)
        nI1 = jnp.where(c1, idx, I1)
        nS2 = jnp.where(c1, S1, jnp.where(c2, v, S2))
        nI2 = jnp.where(c1, I1, jnp.where(c2, idx, I2))
        nS3 = jnp.where(c12, S2, jnp.where(c3, v, S3))
        nI3 = jnp.where(c12, I2, jnp.where(c3, idx, I3))
        nS4 = jnp.where(c123, S3, jnp.where(c4, v, S4))
        nI4 = jnp.where(c123, I3, jnp.where(c4, idx, I4))
        return (nS1, nI1, nS2, nI2, nS3, nI3, nS4, nI4)

    def grp(g, _):
        row_off = g * (L * C) + lanes * C

        def cls(c8, MA):
            M, A = MA
            for dc in range(8):
                c = c8 * 8 + dc
                v = plsc.load_gather(scv, [row_off + c])
                take = v > M
                M = jnp.where(take, v, M)
                A = jnp.where(take, jnp.full((L,), c, jnp.int32), A)
            return (M, A)

        M, A = lax.fori_loop(0, C // 8, cls,
                             (jnp.full((L,), NEG, jnp.float32), zi))
        sl = pl.ds(g * L, L)
        bo4 = (g * L + lanes) * 4
        x1 = plsc.load_gather(bxv, [bo4])
        y1 = plsc.load_gather(bxv, [bo4 + 1])
        x2 = plsc.load_gather(bxv, [bo4 + 2])
        y2 = plsc.load_gather(bxv, [bo4 + 3])
        x1v[sl] = x1
        y1v[sl] = y1
        x2v[sl] = x2
        y2v[sl] = y2
        arv[sl] = (x2 - x1) * (y2 - y1)
        validrow = (rb + g * L + lanes) >= base
        candv[sl] = jnp.where(validrow & (M > THR), M, negv)
        labv[sl] = A
        return 0
    lax.fori_loop(0, NV, grp, 0)

    def cond(st):
        return st[1]

    def body(st):
        k0, _ = st

        def scan(j2, nst):
            for dj in range(2):
                j = j2 * 2 + dj
                nst = top4_update(nst, candv[pl.ds(j * L, L)],
                                  jnp.full((L,), j * L, jnp.int32) + lanes)
            return nst

        t4 = lax.fori_loop(
            0, NV // 2, scan,
            (jnp.full((L,), NEG, jnp.float32), bigv,
             jnp.full((L,), NEG, jnp.float32), bigv,
             jnp.full((L,), NEG, jnp.float32), bigv,
             jnp.full((L,), NEG, jnp.float32), bigv))
        tS = [t4[0], t4[2], t4[4], t4[6]]
        tI = [t4[1], t4[3], t4[5], t4[7]]

        H, HI = tS[0], tI[0]
        dv = jnp.zeros((L,), jnp.int32)
        cM = []
        cL = []
        for i in range(4):
            Mi = jnp.max(H)
            lii = jnp.min(jnp.where(H == Mi, HI, bigv))
            cM.append(Mi)
            cL.append(jnp.clip(lii, 0, ROWS - 1))
            if i < 3:
                match = (H == Mi) & (HI == lii)
                dv = dv + match.astype(jnp.int32)
                nxt = jnp.where(dv == 1, tS[1],
                                jnp.where(dv == 2, tS[2],
                                          jnp.where(dv == 3, tS[3], negv)))
                nxtI = jnp.where(dv == 1, tI[1],
                                 jnp.where(dv == 2, tI[2],
                                           jnp.where(dv == 3, tI[3], bigv)))
                H = jnp.where(match, nxt, H)
                HI = jnp.where(match, nxtI, HI)

        def entry_half(i0):
            lv0 = jnp.full((L,), cL[i0], jnp.int32)
            lv1 = jnp.full((L,), cL[i0 + 1], jnp.int32)
            pk0 = jnp.full((L,), sid * 65536 + cL[i0], jnp.int32)
            pk1 = jnp.full((L,), sid * 65536 + cL[i0 + 1], jnp.int32)
            p = jnp.full((L,), cM[i0], jnp.float32)
            p = jnp.where(lanes == 1, plsc.bitcast(pk0, jnp.float32), p)
            p = jnp.where(lanes == 2, plsc.load_gather(x1v, [lv0]), p)
            p = jnp.where(lanes == 3, plsc.load_gather(y1v, [lv0]), p)
            p = jnp.where(lanes == 4, plsc.load_gather(x2v, [lv0]), p)
            p = jnp.where(lanes == 5, plsc.load_gather(y2v, [lv0]), p)
            p = jnp.where(lanes == 6,
                          plsc.bitcast(plsc.load_gather(labv, [lv0]),
                                       jnp.float32), p)
            p = jnp.where(lanes == 7, plsc.load_gather(arv, [lv0]), p)
            p = jnp.where(lanes == 8, jnp.full((L,), cM[i0 + 1],
                                               jnp.float32), p)
            p = jnp.where(lanes == 9, plsc.bitcast(pk1, jnp.float32), p)
            p = jnp.where(lanes == 10, plsc.load_gather(x1v, [lv1]), p)
            p = jnp.where(lanes == 11, plsc.load_gather(y1v, [lv1]), p)
            p = jnp.where(lanes == 12, plsc.load_gather(x2v, [lv1]), p)
            p = jnp.where(lanes == 13, plsc.load_gather(y2v, [lv1]), p)
            p = jnp.where(lanes == 14,
                          plsc.bitcast(plsc.load_gather(labv, [lv1]),
                                       jnp.float32), p)
            p = jnp.where(lanes == 15, plsc.load_gather(arv, [lv1]), p)
            return p

        pubv[pl.ds(0, L)] = entry_half(0)
        pubv[pl.ds(L, L)] = entry_half(2)

        BW = 2 * L
        par = lax.rem(k0, 2)
        pltpu.sync_copy(pubv, shared.at[pl.ds(par * (NS * BW) + sid * BW,
                                              BW)])
        plsc.subcore_barrier()
        pltpu.sync_copy(shared.at[pl.ds(par * (NS * BW), NS * BW)], rbv)

        def col(c):
            return plsc.load_gather(rbv, [lanes * BW + c])

        s_e = []
        p_e = []
        bx_e = []
        dead0 = []
        for e in range(4):
            s = col(8 * e)
            s_e.append(s)
            p_e.append(plsc.bitcast(col(8 * e + 1), jnp.int32))
            bx_e.append((col(8 * e + 2), col(8 * e + 3),
                         col(8 * e + 4), col(8 * e + 5), col(8 * e + 7)))
            dead0.append(~(s > -1e29))

        def ex_cond(est):
            return est[0]

        def ex_body(est):
            _, k, d0, d1, d2, d3 = est
            dead = [d0, d1, d2, d3]

            ms = [jnp.where(dead[e], negv, s_e[e]) for e in range(4)]
            M = jnp.max(jnp.maximum(jnp.maximum(ms[0], ms[1]),
                                    jnp.maximum(ms[2], ms[3])))
            Mv = jnp.full((L,), M, jnp.float32)
            ps = [jnp.where((~dead[e]) & (s_e[e] == Mv), p_e[e], bigv)
                  for e in range(4)]
            pw = jnp.min(jnp.minimum(jnp.minimum(ps[0], ps[1]),
                                     jnp.minimum(ps[2], ps[3])))
            pwv = jnp.full((L,), pw, jnp.int32)
            got = M > -1e29
            alldead = dead[0] & dead[1] & dead[2] & dead[3]
            outrank = (s_e[3] > Mv) | ((s_e[3] == Mv) & (p_e[3] < pwv))
            viol = jnp.max((alldead & outrank).astype(jnp.int32)) > 0
            accept = got & (~viol) & (k < K)

            is_w = [(~dead[e]) & (s_e[e] == Mv) & (p_e[e] == pwv)
                    for e in range(4)]
            slot = jnp.max((is_w[1].astype(jnp.int32)
                            + 2 * is_w[2].astype(jnp.int32)
                            + 3 * is_w[3].astype(jnp.int32)))

            wsid = jnp.clip(jnp.right_shift(pw, 16), 0, NS - 1)
            wrb = jnp.minimum(wsid * ROWS, N - ROWS)
            srow = wsid * BW + 8 * slot
            sr = jnp.full((L,), srow, jnp.int32)
            X1v = plsc.load_gather(rbv, [sr + 2])
            Y1v = plsc.load_gather(rbv, [sr + 3])
            X2v = plsc.load_gather(rbv, [sr + 4])
            Y2v = plsc.load_gather(rbv, [sr + 5])
            LBv = plsc.bitcast(plsc.load_gather(rbv, [sr + 6]), jnp.int32)
            WAv = plsc.load_gather(rbv, [sr + 7])
            iwv = jnp.full((L,), (pw & 0xFFFF) + wrb, jnp.int32)
            acc_v = jnp.full((L,), accept, jnp.bool_)

            def board_iou(ex1, ey1, ex2, ey2, ear):
                xx1 = jnp.maximum(ex1, X1v)
                yy1 = jnp.maximum(ey1, Y1v)
                xx2 = jnp.minimum(ex2, X2v)
                yy2 = jnp.minimum(ey2, Y2v)
                inter = (jnp.maximum(xx2 - xx1, 0.0) *
                         jnp.maximum(yy2 - yy1, 0.0))
                union = WAv + ear - inter
                return inter / (union + 1e-8)

            dead = [dead[e] | (acc_v & (is_w[e] | (board_iou(*bx_e[e])
                                                   >= IOU_THR)))
                    for e in range(4)]

            @pl.when(accept)
            def _():
                def sweep(j4, _):
                    for dj in range(4):
                        j = j4 * 4 + dj
                        sl = pl.ds(j * L, L)
                        xx1 = jnp.maximum(x1v[sl], X1v)
                        yy1 = jnp.maximum(y1v[sl], Y1v)
                        xx2 = jnp.minimum(x2v[sl], X2v)
                        yy2 = jnp.minimum(y2v[sl], Y2v)
                        inter = (jnp.maximum(xx2 - xx1, 0.0) *
                                 jnp.maximum(yy2 - yy1, 0.0))
                        union = WAv + arv[sl] - inter
                        iou = inter / (union + 1e-8)
                        gi = rb + j * L + lanes
                        kill = (iou >= IOU_THR) | (gi == iwv)
                        candv[sl] = jnp.where(kill, negv, candv[sl])
                    return 0
                lax.fori_loop(0, NV // 4, sweep, 0)

            @pl.when(accept & (sid == 0))
            def _():
                bvals = X1v
                bvals = jnp.where(lanes == 1, Y1v, bvals)
                bvals = jnp.where(lanes == 2, X2v, bvals)
                bvals = jnp.where(lanes == 3, Y2v, bvals)
                plsc.store_scatter(obv, [4 * k + lanes], bvals,
                                   mask=lanes < 4)
                kv = jnp.full((L,), k, jnp.int32)
                plsc.store_scatter(osv, [kv], jnp.full((L,), M, jnp.float32),
                                   mask=lanes == 0)
                plsc.store_scatter(olv, [kv], LBv, mask=lanes == 0)

            k = k + accept.astype(jnp.int32)
            return (accept, k, dead[0], dead[1], dead[2], dead[3])

        est = lax.while_loop(ex_cond, ex_body,
                             (jnp.bool_(True), k0,
                              dead0[0], dead0[1], dead0[2], dead0[3]))
        k1 = est[1]
        cont = (k1 > k0) & (k1 < K)
        return (k1, cont)

    lax.while_loop(cond, body, (jnp.int32(0), jnp.bool_(True)))

    @pl.when(sid == 0)
    def _():
        pltpu.sync_copy(obv, bo_hbm)
        pltpu.sync_copy(osv, so_hbm)
        pltpu.sync_copy(olv, lo_hbm)


@functools.partial(
    pl.kernel,
    out_type=(
        jax.ShapeDtypeStruct((BO_PAD,), jnp.float32),
        jax.ShapeDtypeStruct((SC_PAD,), jnp.float32),
        jax.ShapeDtypeStruct((SC_PAD,), jnp.int32),
    ),
    mesh=plsc.VectorSubcoreMesh(
        core_axis_name="c", subcore_axis_name="s",
        num_cores=1, num_subcores=NS),
    compiler_params=pltpu.CompilerParams(needs_layout_passes=False),
    scratch_types=[
        pltpu.VMEM((ROWS * C,), jnp.float32),
        pltpu.VMEM((ROWS * 4,), jnp.float32),
        pltpu.VMEM((ROWS,), jnp.float32),
        pltpu.VMEM((ROWS,), jnp.float32),
        pltpu.VMEM((ROWS,), jnp.float32),
        pltpu.VMEM((ROWS,), jnp.float32),
        pltpu.VMEM((ROWS,), jnp.float32),
        pltpu.VMEM((ROWS,), jnp.float32),
        pltpu.VMEM((ROWS,), jnp.int32),
        pltpu.VMEM((2 * L,), jnp.float32),
        pltpu.VMEM((NS * 2 * L,), jnp.float32),
        pltpu.VMEM((BO_PAD,), jnp.float32),
        pltpu.VMEM((SC_PAD,), jnp.float32),
        pltpu.VMEM((SC_PAD,), jnp.int32),
        pltpu.SemaphoreType.DMA,
        pltpu.VMEM_SHARED((2 * NS * 2 * L,), jnp.float32),
    ],
)
def _nms_call(sc_hbm, bx_hbm, bo_hbm, so_hbm, lo_hbm, *scratch):
    _nms_kernel(sc_hbm, bx_hbm, bo_hbm, so_hbm, lo_hbm, *scratch)


@jax.jit
def kernel(boxes, scores):
    bo, so, lo = _nms_call(scores.reshape(-1), boxes.reshape(-1))
    return (bo[:4 * K].reshape(1, K, 4), so[:K][None], lo[:K][None])
